# baseline (speedup 1.0000x reference)
import functools

import jax
import jax.numpy as jnp
from jax import lax
from jax.experimental import pallas as pl
from jax.experimental.pallas import tpu as pltpu
from jax.experimental.pallas import tpu_sc as plsc

NC = 2
NS = 16
L = 16
NW = NC * NS

B = 16384
D = 64
BPW = B // NW
CHUNK = 128
NCHUNK = BPW // CHUNK
CGROUPS = CHUNK // L

N_ENT = 1000000
N_REL = 1000
BE = 4096
SEG_ENT = 62 * BE
BE_REL = 256
SEG_REL = BE_REL


def _relayout(table_t, seg, be):
    n = table_t.shape[1]
    grid = seg // be
    jmax = (n - be) // be
    lf3 = (n - 3 * seg) // be
    tail = jnp.pad(
        table_t[:, 3 * seg + lf3 * be:],
        ((0, 0), (0, 3 * seg + (lf3 + 1) * be - n)),
    )

    def bits16(t):
        return lax.bitcast_convert_type(
            t.astype(jnp.bfloat16), jnp.uint16).astype(jnp.uint32)

    def body(x0ref, x1ref, x2ref, x3ref, tref, oref):
        i = pl.program_id(0)
        b0 = bits16(x0ref[...].T)
        b1 = bits16(x1ref[...].T)
        b2 = bits16(x2ref[...].T)
        x3 = jnp.where(i < lf3, x3ref[...], tref[...])
        b3 = bits16(x3.T)
        w01 = jnp.bitwise_or(jnp.left_shift(b1, 16), b0)
        w23 = jnp.bitwise_or(jnp.left_shift(b3, 16), b2)
        oref[...] = lax.bitcast_convert_type(
            jnp.concatenate([w01, w23], axis=1), jnp.int32)

    in_specs = [
        pl.BlockSpec((D, be), lambda i, q=q, g=grid, jm=jmax:
                     (0, jnp.minimum(q * g + i, jm)))
        for q in range(4)
    ]
    in_specs.append(pl.BlockSpec((D, be), lambda i: (0, 0)))
    return pl.pallas_call(
        body,
        grid=(grid,),
        in_specs=in_specs,
        out_specs=pl.BlockSpec((be, 2 * D), lambda i: (i, 0)),
        out_shape=jax.ShapeDtypeStruct((seg, 2 * D), jnp.int32),
    )(table_t, table_t, table_t, table_t, tail)


@functools.partial(
    pl.kernel,
    out_type=jax.ShapeDtypeStruct((B,), jnp.float32),
    mesh=plsc.VectorSubcoreMesh(core_axis_name="c", subcore_axis_name="s"),
    compiler_params=pltpu.CompilerParams(
        needs_layout_passes=False, use_tc_tiling_on_sc=True),
    scratch_types=[
        pltpu.VMEM((BPW,), jnp.int32),
        pltpu.VMEM((BPW,), jnp.int32),
        pltpu.VMEM((BPW,), jnp.int32),
        pltpu.VMEM((BPW,), jnp.int32),
        pltpu.VMEM((BPW,), jnp.int32),
        pltpu.VMEM((BPW,), jnp.int32),
        pltpu.VMEM((BPW,), jnp.int32),
        pltpu.VMEM((BPW,), jnp.int32),
        pltpu.VMEM((BPW,), jnp.int32),
        pltpu.VMEM((CHUNK, 2 * D), jnp.int32),
        pltpu.VMEM((CHUNK, 2 * D), jnp.int32),
        pltpu.VMEM((CHUNK, 2 * D), jnp.int32),
        pltpu.VMEM((CHUNK, 2 * D), jnp.int32),
        pltpu.VMEM((SEG_REL, 2 * D), jnp.int32),
        pltpu.VMEM((BPW,), jnp.float32),
        pltpu.SemaphoreType.DMA,
        pltpu.SemaphoreType.DMA,
    ],
)
def _distmult_sc(heads_hbm, relations_hbm, tails_hbm, ent_hbm, rel_hbm,
                 out_hbm, hidx, ridx, tidx, hrow, rrow, trow, hseg, rseg, tseg,
                 e1v0, e2v0, e1v1, e2v1, relv, outv, sem0, sem1):
    wid = lax.axis_index("s") * NC + lax.axis_index("c")
    base = wid * BPW

    pltpu.sync_copy(heads_hbm.at[pl.ds(base, BPW)], hidx)
    pltpu.sync_copy(relations_hbm.at[pl.ds(base, BPW)], ridx)
    pltpu.sync_copy(tails_hbm.at[pl.ds(base, BPW)], tidx)

    def seg_of(x, seg):
        one = jnp.int32(1)
        zero = jnp.int32(0)
        return (jnp.where(x >= seg, one, zero)
                + jnp.where(x >= 2 * seg, one, zero)
                + jnp.where(x >= 3 * seg, one, zero))

    def fold(i, carry):
        s = pl.ds(i * L, L)
        h = hidx[s]
        r = ridx[s]
        t = tidx[s]
        hq = seg_of(h, SEG_ENT)
        rq = seg_of(r, SEG_REL)
        tq = seg_of(t, SEG_ENT)
        hseg[s] = hq
        rseg[s] = rq
        tseg[s] = tq
        hrow[s] = h - hq * SEG_ENT
        rrow[s] = r - rq * SEG_REL
        trow[s] = t - tq * SEG_ENT
        return carry

    lax.fori_loop(0, BPW // L, fold, 0)

    bufs = ((e1v0, e2v0, sem0), (e1v1, e2v1, sem1))

    def issue(c, bi):
        cbase = c * CHUNK
        s = pl.ds(cbase, CHUNK)
        b = bufs[bi]
        return (
            pltpu.async_copy(ent_hbm.at[hrow.at[s]], b[0], b[2]),
            pltpu.async_copy(ent_hbm.at[trow.at[s]], b[1], b[2]),
        )

    def compute(c, bi):
        cbase = c * CHUNK
        e1v, e2v, _ = bufs[bi]

        def group(g, gcarry):
            rows = g * L + lax.iota(jnp.int32, L)
            s = pl.ds(cbase + g * L, L)
            hq = hseg[s]
            rq = rseg[s]
            tq = tseg[s]
            rrows = rrow[s]
            cb_h = (hq & 2) * 32
            cb_r = (rq & 2) * 32
            cb_t = (tq & 2) * 32
            hi_h = (hq & 1) > 0
            hi_r = (rq & 1) > 0
            hi_t = (tq & 1) > 0
            himask = jnp.int32(-65536)

            def extract(w, hi):
                bits = jnp.where(hi, w & himask, jnp.left_shift(w, 16))
                return plsc.bitcast(bits, jnp.float32)

            acc_d = jnp.zeros((L,), jnp.float32)
            acc_n1 = jnp.zeros((L,), jnp.float32)
            acc_n2 = jnp.zeros((L,), jnp.float32)
            for k in range(D):
                a = extract(plsc.load_gather(e1v, [rows, cb_h + k]), hi_h)
                r_ = extract(plsc.load_gather(relv, [rrows, cb_r + k]), hi_r)
                b = extract(plsc.load_gather(e2v, [rows, cb_t + k]), hi_t)
                acc_d = acc_d + a * r_ * b
                acc_n1 = acc_n1 + a * a
                acc_n2 = acc_n2 + b * b
            inv1 = _rsqrt(jnp.maximum(acc_n1, 1e-24))
            inv2 = _rsqrt(jnp.maximum(acc_n2, 1e-24))
            outv[s] = acc_d * inv1 * inv2
            return gcarry

        lax.fori_loop(0, CGROUPS, group, 0)

    pltpu.sync_copy(rel_hbm, relv)

    descs = issue(0, 0)
    for c in range(NCHUNK):
        bi = c % 2
        if c + 1 < NCHUNK:
            nxt = issue(c + 1, 1 - bi)
        for d in descs:
            d.wait()
        compute(c, bi)
        if c + 1 < NCHUNK:
            descs = nxt

    pltpu.sync_copy(outv, out_hbm.at[pl.ds(base, BPW)])


def _rsqrt(x):
    i = plsc.bitcast(x, jnp.int32)
    i = jnp.int32(0x5F3759DF) - lax.shift_right_logical(i, 1)
    y = plsc.bitcast(i, jnp.float32)
    for _ in range(3):
        y = y * (1.5 - 0.5 * x * y * y)
    return y


def kernel(heads, relations, tails, entity_embedding, relation_embedding):
    ent2 = _relayout(entity_embedding.T, SEG_ENT, BE)
    rel2 = _relayout(relation_embedding.T, SEG_REL, BE_REL)
    return _distmult_sc(
        heads.astype(jnp.int32),
        relations.astype(jnp.int32),
        tails.astype(jnp.int32),
        ent2,
        rel2,
    )


# speedup vs baseline: 1.0720x; 1.0720x over previous
import functools

import jax
import jax.numpy as jnp
from jax import lax
from jax.experimental import pallas as pl
from jax.experimental.pallas import tpu as pltpu
from jax.experimental.pallas import tpu_sc as plsc

NC = 2
NS = 16
L = 16
NW = NC * NS

B = 16384
D = 64
BPW = B // NW
CHUNK = 128
NCHUNK = BPW // CHUNK
CGROUPS = CHUNK // L

N_ENT = 1000000
N_REL = 1000
BE = 8192
SEG_ENT = 31 * BE
BE_REL = 256
SEG_REL = BE_REL


def _relayout(table_t, seg, be):
    n = table_t.shape[1]
    grid = seg // be
    jmax = (n - be) // be
    lf3 = (n - 3 * seg) // be
    tail = jnp.pad(
        table_t[:, 3 * seg + lf3 * be:],
        ((0, 0), (0, 3 * seg + (lf3 + 1) * be - n)),
    )

    def bits16(t):
        return lax.bitcast_convert_type(
            t.astype(jnp.bfloat16), jnp.uint16).astype(jnp.uint32)

    def body(x0ref, x1ref, x2ref, x3ref, tref, oref):
        i = pl.program_id(0)
        b0 = bits16(x0ref[...].T)
        b1 = bits16(x1ref[...].T)
        b2 = bits16(x2ref[...].T)
        x3 = jnp.where(i < lf3, x3ref[...], tref[...])
        b3 = bits16(x3.T)
        w01 = jnp.bitwise_or(jnp.left_shift(b1, 16), b0)
        w23 = jnp.bitwise_or(jnp.left_shift(b3, 16), b2)
        oref[...] = lax.bitcast_convert_type(
            jnp.concatenate([w01, w23], axis=1), jnp.int32)

    in_specs = [
        pl.BlockSpec((D, be), lambda i, q=q, g=grid, jm=jmax:
                     (0, jnp.minimum(q * g + i, jm)))
        for q in range(4)
    ]
    in_specs.append(pl.BlockSpec((D, be), lambda i: (0, 0)))
    return pl.pallas_call(
        body,
        grid=(grid,),
        in_specs=in_specs,
        out_specs=pl.BlockSpec((be, 2 * D), lambda i: (i, 0)),
        out_shape=jax.ShapeDtypeStruct((seg, 2 * D), jnp.int32),
    )(table_t, table_t, table_t, table_t, tail)


@functools.partial(
    pl.kernel,
    out_type=jax.ShapeDtypeStruct((B,), jnp.float32),
    mesh=plsc.VectorSubcoreMesh(core_axis_name="c", subcore_axis_name="s"),
    compiler_params=pltpu.CompilerParams(
        needs_layout_passes=False, use_tc_tiling_on_sc=True),
    scratch_types=[
        pltpu.VMEM((BPW,), jnp.int32),
        pltpu.VMEM((BPW,), jnp.int32),
        pltpu.VMEM((BPW,), jnp.int32),
        pltpu.VMEM((BPW,), jnp.int32),
        pltpu.VMEM((BPW,), jnp.int32),
        pltpu.VMEM((BPW,), jnp.int32),
        pltpu.VMEM((BPW,), jnp.int32),
        pltpu.VMEM((BPW,), jnp.int32),
        pltpu.VMEM((BPW,), jnp.int32),
        pltpu.VMEM((CHUNK, 2 * D), jnp.int32),
        pltpu.VMEM((CHUNK, 2 * D), jnp.int32),
        pltpu.VMEM((CHUNK, 2 * D), jnp.int32),
        pltpu.VMEM((CHUNK, 2 * D), jnp.int32),
        pltpu.VMEM((SEG_REL, 2 * D), jnp.int32),
        pltpu.VMEM((BPW,), jnp.float32),
        pltpu.SemaphoreType.DMA,
        pltpu.SemaphoreType.DMA,
    ],
)
def _distmult_sc(heads_hbm, relations_hbm, tails_hbm, ent_hbm, rel_hbm,
                 out_hbm, hidx, ridx, tidx, hrow, rrow, trow, hseg, rseg, tseg,
                 e1v0, e2v0, e1v1, e2v1, relv, outv, sem0, sem1):
    wid = lax.axis_index("s") * NC + lax.axis_index("c")
    base = wid * BPW

    pltpu.sync_copy(heads_hbm.at[pl.ds(base, BPW)], hidx)
    pltpu.sync_copy(relations_hbm.at[pl.ds(base, BPW)], ridx)
    pltpu.sync_copy(tails_hbm.at[pl.ds(base, BPW)], tidx)

    def seg_of(x, seg):
        one = jnp.int32(1)
        zero = jnp.int32(0)
        return (jnp.where(x >= seg, one, zero)
                + jnp.where(x >= 2 * seg, one, zero)
                + jnp.where(x >= 3 * seg, one, zero))

    def fold(i, carry):
        s = pl.ds(i * L, L)
        h = hidx[s]
        r = ridx[s]
        t = tidx[s]
        hq = seg_of(h, SEG_ENT)
        rq = seg_of(r, SEG_REL)
        tq = seg_of(t, SEG_ENT)
        hseg[s] = hq
        rseg[s] = rq
        tseg[s] = tq
        hrow[s] = h - hq * SEG_ENT
        rrow[s] = r - rq * SEG_REL
        trow[s] = t - tq * SEG_ENT
        return carry

    lax.fori_loop(0, BPW // L, fold, 0)

    bufs = ((e1v0, e2v0, sem0), (e1v1, e2v1, sem1))

    def issue(c, bi):
        cbase = c * CHUNK
        s = pl.ds(cbase, CHUNK)
        b = bufs[bi]
        return (
            pltpu.async_copy(ent_hbm.at[hrow.at[s]], b[0], b[2]),
            pltpu.async_copy(ent_hbm.at[trow.at[s]], b[1], b[2]),
        )

    def compute(c, bi):
        cbase = c * CHUNK
        e1v, e2v, _ = bufs[bi]

        def group(g, gcarry):
            rows = g * L + lax.iota(jnp.int32, L)
            s = pl.ds(cbase + g * L, L)
            hq = hseg[s]
            rq = rseg[s]
            tq = tseg[s]
            rrows = rrow[s]
            cb_h = (hq & 2) * 32
            cb_r = (rq & 2) * 32
            cb_t = (tq & 2) * 32
            hi_h = (hq & 1) > 0
            hi_r = (rq & 1) > 0
            hi_t = (tq & 1) > 0
            himask = jnp.int32(-65536)

            def extract(w, hi):
                bits = jnp.where(hi, w & himask, jnp.left_shift(w, 16))
                return plsc.bitcast(bits, jnp.float32)

            acc_d = jnp.zeros((L,), jnp.float32)
            acc_n1 = jnp.zeros((L,), jnp.float32)
            acc_n2 = jnp.zeros((L,), jnp.float32)
            for k in range(D):
                a = extract(plsc.load_gather(e1v, [rows, cb_h + k]), hi_h)
                r_ = extract(plsc.load_gather(relv, [rrows, cb_r + k]), hi_r)
                b = extract(plsc.load_gather(e2v, [rows, cb_t + k]), hi_t)
                acc_d = acc_d + a * r_ * b
                acc_n1 = acc_n1 + a * a
                acc_n2 = acc_n2 + b * b
            inv1 = _rsqrt(jnp.maximum(acc_n1, 1e-24))
            inv2 = _rsqrt(jnp.maximum(acc_n2, 1e-24))
            outv[s] = acc_d * inv1 * inv2
            return gcarry

        lax.fori_loop(0, CGROUPS, group, 0)

    pltpu.sync_copy(rel_hbm, relv)

    descs = issue(0, 0)
    for c in range(NCHUNK):
        bi = c % 2
        if c + 1 < NCHUNK:
            nxt = issue(c + 1, 1 - bi)
        for d in descs:
            d.wait()
        compute(c, bi)
        if c + 1 < NCHUNK:
            descs = nxt

    pltpu.sync_copy(outv, out_hbm.at[pl.ds(base, BPW)])


def _rsqrt(x):
    i = plsc.bitcast(x, jnp.int32)
    i = jnp.int32(0x5F3759DF) - lax.shift_right_logical(i, 1)
    y = plsc.bitcast(i, jnp.float32)
    for _ in range(3):
        y = y * (1.5 - 0.5 * x * y * y)
    return y


def kernel(heads, relations, tails, entity_embedding, relation_embedding):
    ent2 = _relayout(entity_embedding.T, SEG_ENT, BE)
    rel2 = _relayout(relation_embedding.T, SEG_REL, BE_REL)
    return _distmult_sc(
        heads.astype(jnp.int32),
        relations.astype(jnp.int32),
        tails.astype(jnp.int32),
        ent2,
        rel2,
    )


# speedup vs baseline: 1.0833x; 1.0106x over previous
import functools

import jax
import jax.numpy as jnp
from jax import lax
from jax.experimental import pallas as pl
from jax.experimental.pallas import tpu as pltpu
from jax.experimental.pallas import tpu_sc as plsc

NC = 2
NS = 16
L = 16
NW = NC * NS

B = 16384
D = 64
BPW = B // NW
CHUNK = 128
NCHUNK = BPW // CHUNK
CGROUPS = CHUNK // L

N_ENT = 1000000
N_REL = 1000
BE = 8192
SEG_ENT = 31 * BE
BE_REL = 256
SEG_REL = BE_REL


def _relayout(table_t, seg, be):
    n = table_t.shape[1]
    grid = seg // be
    jmax = (n - be) // be
    lf3 = (n - 3 * seg) // be
    tail = jnp.pad(
        table_t[:, 3 * seg + lf3 * be:],
        ((0, 0), (0, 3 * seg + (lf3 + 1) * be - n)),
    )

    def bits16(t):
        return lax.bitcast_convert_type(
            t.astype(jnp.bfloat16), jnp.uint16).astype(jnp.uint32)

    def body(x0ref, x1ref, x2ref, x3ref, tref, oref):
        i = pl.program_id(0)
        b0 = bits16(x0ref[...].T)
        b1 = bits16(x1ref[...].T)
        b2 = bits16(x2ref[...].T)
        x3 = jnp.where(i < lf3, x3ref[...], tref[...])
        b3 = bits16(x3.T)
        w01 = jnp.bitwise_or(jnp.left_shift(b1, 16), b0)
        w23 = jnp.bitwise_or(jnp.left_shift(b3, 16), b2)
        oref[...] = lax.bitcast_convert_type(
            jnp.concatenate([w01, w23], axis=1), jnp.int32)

    in_specs = [
        pl.BlockSpec((D, be), lambda i, q=q, g=grid, jm=jmax:
                     (0, jnp.minimum(q * g + i, jm)))
        for q in range(4)
    ]
    in_specs.append(pl.BlockSpec((D, be), lambda i: (0, 0)))
    return pl.pallas_call(
        body,
        grid=(grid,),
        in_specs=in_specs,
        out_specs=pl.BlockSpec((be, 2 * D), lambda i: (i, 0)),
        out_shape=jax.ShapeDtypeStruct((seg, 2 * D), jnp.int32),
    )(table_t, table_t, table_t, table_t, tail)


@functools.partial(
    pl.kernel,
    out_type=jax.ShapeDtypeStruct((B,), jnp.float32),
    mesh=plsc.VectorSubcoreMesh(core_axis_name="c", subcore_axis_name="s"),
    compiler_params=pltpu.CompilerParams(
        needs_layout_passes=False, use_tc_tiling_on_sc=True),
    scratch_types=[
        pltpu.VMEM((BPW,), jnp.int32),
        pltpu.VMEM((BPW,), jnp.int32),
        pltpu.VMEM((BPW,), jnp.int32),
        pltpu.VMEM((BPW,), jnp.int32),
        pltpu.VMEM((BPW,), jnp.int32),
        pltpu.VMEM((BPW,), jnp.int32),
        pltpu.VMEM((BPW,), jnp.int32),
        pltpu.VMEM((BPW,), jnp.int32),
        pltpu.VMEM((BPW,), jnp.int32),
        pltpu.VMEM((CHUNK, 2 * D), jnp.int32),
        pltpu.VMEM((CHUNK, 2 * D), jnp.int32),
        pltpu.VMEM((CHUNK, 2 * D), jnp.int32),
        pltpu.VMEM((CHUNK, 2 * D), jnp.int32),
        pltpu.VMEM((SEG_REL, 2 * D), jnp.int32),
        pltpu.VMEM((BPW,), jnp.float32),
        pltpu.SemaphoreType.DMA,
        pltpu.SemaphoreType.DMA,
    ],
)
def _distmult_sc(heads_hbm, relations_hbm, tails_hbm, ent_hbm, rel_hbm,
                 out_hbm, hidx, ridx, tidx, hrow, rrow, trow, hseg, rseg, tseg,
                 e1v0, e2v0, e1v1, e2v1, relv, outv, sem0, sem1):
    wid = lax.axis_index("s") * NC + lax.axis_index("c")
    base = wid * BPW

    pltpu.sync_copy(heads_hbm.at[pl.ds(base, BPW)], hidx)
    pltpu.sync_copy(relations_hbm.at[pl.ds(base, BPW)], ridx)
    pltpu.sync_copy(tails_hbm.at[pl.ds(base, BPW)], tidx)

    def seg_of(x, seg):
        one = jnp.int32(1)
        zero = jnp.int32(0)
        return (jnp.where(x >= seg, one, zero)
                + jnp.where(x >= 2 * seg, one, zero)
                + jnp.where(x >= 3 * seg, one, zero))

    def fold(i, carry):
        s = pl.ds(i * L, L)
        h = hidx[s]
        r = ridx[s]
        t = tidx[s]
        hq = seg_of(h, SEG_ENT)
        rq = seg_of(r, SEG_REL)
        tq = seg_of(t, SEG_ENT)
        hseg[s] = hq
        rseg[s] = rq
        tseg[s] = tq
        hrow[s] = h - hq * SEG_ENT
        rrow[s] = r - rq * SEG_REL
        trow[s] = t - tq * SEG_ENT
        return carry

    lax.fori_loop(0, BPW // L, fold, 0)

    bufs = ((e1v0, e2v0, sem0), (e1v1, e2v1, sem1))

    def issue(c, bi):
        cbase = c * CHUNK
        s = pl.ds(cbase, CHUNK)
        b = bufs[bi]
        return (
            pltpu.async_copy(ent_hbm.at[hrow.at[s]], b[0], b[2]),
            pltpu.async_copy(ent_hbm.at[trow.at[s]], b[1], b[2]),
        )

    def compute(c, bi):
        cbase = c * CHUNK
        e1v, e2v, _ = bufs[bi]

        def group(g, gcarry):
            rows = g * L + lax.iota(jnp.int32, L)
            s = pl.ds(cbase + g * L, L)
            hq = hseg[s]
            rq = rseg[s]
            tq = tseg[s]
            rrows = rrow[s]
            cb_h = (hq & 2) * 32
            cb_r = (rq & 2) * 32
            cb_t = (tq & 2) * 32
            sh_h = jnp.where((hq & 1) > 0, 0, 16)
            sh_r = jnp.where((rq & 1) > 0, 0, 16)
            sh_t = jnp.where((tq & 1) > 0, 0, 16)
            himask = jnp.int32(-65536)

            def extract(w, sh):
                bits = jnp.left_shift(w, sh) & himask
                return plsc.bitcast(bits, jnp.float32)

            acc_d = jnp.zeros((L,), jnp.float32)
            acc_n1 = jnp.zeros((L,), jnp.float32)
            acc_n2 = jnp.zeros((L,), jnp.float32)
            for k in range(D):
                a = extract(plsc.load_gather(e1v, [rows, cb_h + k]), sh_h)
                r_ = extract(plsc.load_gather(relv, [rrows, cb_r + k]), sh_r)
                b = extract(plsc.load_gather(e2v, [rows, cb_t + k]), sh_t)
                acc_d = acc_d + a * r_ * b
                acc_n1 = acc_n1 + a * a
                acc_n2 = acc_n2 + b * b
            inv1 = _rsqrt(jnp.maximum(acc_n1, 1e-24))
            inv2 = _rsqrt(jnp.maximum(acc_n2, 1e-24))
            outv[s] = acc_d * inv1 * inv2
            return gcarry

        lax.fori_loop(0, CGROUPS, group, 0)

    pltpu.sync_copy(rel_hbm, relv)

    descs = issue(0, 0)
    for c in range(NCHUNK):
        bi = c % 2
        if c + 1 < NCHUNK:
            nxt = issue(c + 1, 1 - bi)
        for d in descs:
            d.wait()
        compute(c, bi)
        if c + 1 < NCHUNK:
            descs = nxt

    pltpu.sync_copy(outv, out_hbm.at[pl.ds(base, BPW)])


def _rsqrt(x):
    i = plsc.bitcast(x, jnp.int32)
    i = jnp.int32(0x5F3759DF) - lax.shift_right_logical(i, 1)
    y = plsc.bitcast(i, jnp.float32)
    for _ in range(3):
        y = y * (1.5 - 0.5 * x * y * y)
    return y


def kernel(heads, relations, tails, entity_embedding, relation_embedding):
    ent2 = _relayout(entity_embedding.T, SEG_ENT, BE)
    rel2 = _relayout(relation_embedding.T, SEG_REL, BE_REL)
    return _distmult_sc(
        heads.astype(jnp.int32),
        relations.astype(jnp.int32),
        tails.astype(jnp.int32),
        ent2,
        rel2,
    )
